# baseline (speedup 1.0000x reference)
import functools

import jax
import jax.numpy as jnp
from jax import lax
from jax.experimental import pallas as pl
from jax.experimental.pallas import tpu as pltpu
from jax.experimental.pallas import tpu_sc as plsc

_NC = 2
_NS = 16
_LANES = 16
_BBLK = 16


def _combine_sc(xs, gbc):
    ps, bb, cc, ll, dd = xs.shape
    bpw = bb // (_NC * _NS)
    nk = dd // _LANES
    mesh = plsc.VectorSubcoreMesh(core_axis_name="c", subcore_axis_name="s")

    @functools.partial(
        pl.kernel,
        out_type=jax.ShapeDtypeStruct((bb, cc, dd), jnp.float32),
        mesh=mesh,
        scratch_types=[
            pltpu.VMEM((ps, cc, dd), jnp.float32),
            pltpu.VMEM((ps, _LANES), jnp.float32),
            pltpu.VMEM((cc, dd), jnp.float32),
        ],
        compiler_params=pltpu.CompilerParams(use_tc_tiling_on_sc=True),
    )
    def k(xs_hbm, gbc_hbm, comb_hbm, xbuf, gbuf, obuf):
        wid = lax.axis_index("s") * _NC + lax.axis_index("c")

        def do_b(j, carry):
            bidx = wid * bpw + j
            for i in range(ps):
                pltpu.sync_copy(xs_hbm.at[i, bidx, :, ll - 1, :], xbuf.at[i])
            pltpu.sync_copy(gbc_hbm.at[bidx], gbuf)
            g = [jnp.maximum(gbuf[i], 0.0) for i in range(ps)]

            def do_c(c, carry2):
                def do_k(kk, carry3):
                    sl = pl.ds(kk * _LANES, _LANES)
                    acc = xbuf[0, c, sl] * g[0] + 1e-9
                    for i in range(1, ps):
                        acc = acc + xbuf[i, c, sl] * g[i]
                    obuf[c, sl] = acc
                    return carry3

                return lax.fori_loop(0, nk, do_k, carry2)

            lax.fori_loop(0, cc, do_c, 0)
            pltpu.sync_copy(obuf, comb_hbm.at[bidx])
            return carry

        lax.fori_loop(0, bpw, do_b, 0)

    return k(xs, gbc)


def _head_kernel(x_ref, wt_ref, b_ref, o_ref):
    comb = x_ref[...]
    res = jax.lax.dot_general(
        comb, wt_ref[...], (((2,), (0,)), ((), ())),
        preferred_element_type=jnp.float32)
    res = res + b_ref[0][None, None, :]
    o_ref[...] = jnp.transpose(res, (0, 2, 1))


def _head_tc(comb, wt, b2):
    bb, cc, dd = comb.shape
    pred = wt.shape[1]
    grid = (bb // _BBLK,)
    return pl.pallas_call(
        _head_kernel,
        grid=grid,
        in_specs=[
            pl.BlockSpec((_BBLK, cc, dd), lambda t: (t, 0, 0)),
            pl.BlockSpec((dd, pred), lambda t: (0, 0)),
            pl.BlockSpec((1, pred), lambda t: (0, 0)),
        ],
        out_specs=pl.BlockSpec((_BBLK, pred, cc), lambda t: (t, 0, 0)),
        out_shape=jax.ShapeDtypeStruct((bb, pred, cc), jnp.float32),
    )(comb, wt, b2)


def kernel(xs, gates, W, b):
    ps, bb, cc, ll, dd = xs.shape
    pred = W.shape[0]
    gbc = jnp.broadcast_to(gates[:, :, None], (bb, ps, _LANES))
    comb = _combine_sc(xs, gbc)
    return _head_tc(comb, W.T, b.reshape(1, pred))


# speedup vs baseline: 1.1843x; 1.1843x over previous
import functools

import jax
import jax.numpy as jnp
from jax import lax
from jax.experimental import pallas as pl
from jax.experimental.pallas import tpu as pltpu
from jax.experimental.pallas import tpu_sc as plsc

_NC = 2
_NS = 16
_LANES = 16
_BBLK = 16


def _combine_sc(xs, gbc):
    ps, bb, cc, ll, dd = xs.shape
    bpw = bb // (_NC * _NS)
    nk = dd // _LANES
    mesh = plsc.VectorSubcoreMesh(core_axis_name="c", subcore_axis_name="s")

    @functools.partial(
        pl.kernel,
        out_type=jax.ShapeDtypeStruct((bb, cc, dd), jnp.float32),
        mesh=mesh,
        scratch_types=[
            pltpu.VMEM((2, ps, cc, dd), jnp.float32),
            pltpu.VMEM((bpw, ps, _LANES), jnp.float32),
            pltpu.VMEM((2, cc, dd), jnp.float32),
            pltpu.SemaphoreType.DMA((2,)),
            pltpu.SemaphoreType.DMA((2,)),
        ],
        compiler_params=pltpu.CompilerParams(use_tc_tiling_on_sc=True),
    )
    def k(xs_hbm, gbc_hbm, comb_hbm, xbuf, gbuf, obuf, insem, outsem):
        wid = lax.axis_index("s") * _NC + lax.axis_index("c")
        b0 = wid * bpw

        def in_copy(slot, j, i):
            return pltpu.make_async_copy(
                xs_hbm.at[i, b0 + j, :, ll - 1, :], xbuf.at[slot, i],
                insem.at[slot])

        def out_copy(slot, j):
            return pltpu.make_async_copy(
                obuf.at[slot], comb_hbm.at[b0 + j], outsem.at[slot])

        pltpu.sync_copy(gbc_hbm.at[pl.ds(b0, bpw)], gbuf)
        for i in range(ps):
            in_copy(0, 0, i).start()
        for i in range(ps):
            in_copy(1, 1, i).start()

        for j in range(bpw):
            s = j % 2
            for i in range(ps):
                in_copy(s, j, i).wait()
            if j >= 2:
                out_copy(s, j - 2).wait()
            g = [jnp.maximum(gbuf[j, i], 0.0) for i in range(ps)]

            def do_c(c, carry, s=s, g=g):
                for kk in range(nk):
                    sl = pl.ds(kk * _LANES, _LANES)
                    acc = xbuf[s, 0, c, sl] * g[0] + 1e-9
                    for i in range(1, ps):
                        acc = acc + xbuf[s, i, c, sl] * g[i]
                    obuf[s, c, sl] = acc
                return carry

            lax.fori_loop(0, cc, do_c, 0)
            out_copy(s, j).start()
            if j + 2 < bpw:
                for i in range(ps):
                    in_copy(s, j + 2, i).start()

        out_copy((bpw - 2) % 2, bpw - 2).wait()
        out_copy((bpw - 1) % 2, bpw - 1).wait()

    return k(xs, gbc)


def _head_kernel(x_ref, wt_ref, b_ref, o_ref):
    comb = x_ref[...]
    res = jax.lax.dot_general(
        comb, wt_ref[...], (((2,), (0,)), ((), ())),
        preferred_element_type=jnp.float32)
    res = res + b_ref[0][None, None, :]
    o_ref[...] = jnp.transpose(res, (0, 2, 1))


def _head_tc(comb, wt, b2):
    bb, cc, dd = comb.shape
    pred = wt.shape[1]
    grid = (bb // _BBLK,)
    return pl.pallas_call(
        _head_kernel,
        grid=grid,
        in_specs=[
            pl.BlockSpec((_BBLK, cc, dd), lambda t: (t, 0, 0)),
            pl.BlockSpec((dd, pred), lambda t: (0, 0)),
            pl.BlockSpec((1, pred), lambda t: (0, 0)),
        ],
        out_specs=pl.BlockSpec((_BBLK, pred, cc), lambda t: (t, 0, 0)),
        out_shape=jax.ShapeDtypeStruct((bb, pred, cc), jnp.float32),
    )(comb, wt, b2)


def kernel(xs, gates, W, b):
    ps, bb, cc, ll, dd = xs.shape
    pred = W.shape[0]
    gbc = jnp.broadcast_to(gates[:, :, None], (bb, ps, _LANES))
    comb = _combine_sc(xs, gbc)
    return _head_tc(comb, W.T, b.reshape(1, pred))
